# baseline (speedup 1.0000x reference)
import jax
import jax.numpy as jnp
from jax.experimental import pallas as pl
from jax.experimental.pallas import tpu as pltpu

_R = 512
_KA = 136
_LOG2E = 1.4426950408889634
_LN2 = 0.6931471805599453


def _logsoftmax_kernel(x_ref, w_ref, a_ref, o_ref):
    xb = x_ref[0]
    l2 = jnp.dot(w_ref[...], xb, preferred_element_type=jnp.float32)
    xf = xb.astype(jnp.float32)
    n2 = jnp.sum(xf * xf, axis=0, keepdims=True)
    c = a_ref[0, 0] * jnp.sqrt(n2) + 0.5
    s = jnp.sum(jnp.exp2(l2 - c), axis=0, keepdims=True)
    m2 = c + jnp.log2(s)
    o_ref[0] = (l2 - m2) * _LN2


@jax.jit
def kernel(x, xl, W, b):
    B, T, D = x.shape
    V = W.shape[0]
    xb16 = x.astype(jnp.bfloat16).transpose(0, 2, 1)
    ones = jnp.ones((B, 1, T), dtype=jnp.bfloat16)
    zeros = jnp.zeros((B, _KA - D - 1, T), dtype=jnp.bfloat16)
    xt = jnp.concatenate([xb16, ones, zeros], axis=1)
    wb16 = (W * _LOG2E).astype(jnp.bfloat16)
    bcol = (b * _LOG2E).astype(jnp.bfloat16).reshape(V, 1)
    wzeros = jnp.zeros((V, _KA - D - 1), dtype=jnp.bfloat16)
    wa = jnp.concatenate([wb16, bcol, wzeros], axis=1)
    wf = wa.astype(jnp.float32)
    a_max = jnp.max(jnp.sqrt(jnp.sum(wf * wf, axis=1))).reshape(1, 1)

    out_t = pl.pallas_call(
        _logsoftmax_kernel,
        grid=(B, T // _R),
        in_specs=[
            pl.BlockSpec((1, _KA, _R), lambda bi, ti: (bi, 0, ti)),
            pl.BlockSpec((V, _KA), lambda bi, ti: (0, 0)),
            pl.BlockSpec((1, 1), lambda bi, ti: (0, 0)),
        ],
        out_specs=pl.BlockSpec((1, V, _R), lambda bi, ti: (bi, 0, ti)),
        out_shape=jax.ShapeDtypeStruct((B, V, T), jnp.float32),
        compiler_params=pltpu.CompilerParams(
            dimension_semantics=("parallel", "parallel")
        ),
    )(xt, wa, a_max)
    return out_t.transpose(0, 2, 1)


# speedup vs baseline: 1.0672x; 1.0672x over previous
import jax
import jax.numpy as jnp
from jax.experimental import pallas as pl

_R = 512
_KA = 136
_LOG2E = 1.4426950408889634
_LN2 = 0.6931471805599453


def _logsoftmax_kernel(x_ref, w_ref, a_ref, o_ref):
    xb = x_ref[0]
    l2 = jnp.dot(w_ref[...], xb, preferred_element_type=jnp.float32)
    xf = xb.astype(jnp.float32)
    n2 = jnp.sum(xf * xf, axis=0, keepdims=True)
    c = a_ref[0, 0] * jnp.sqrt(n2) + 0.5
    s = jnp.sum(jnp.exp2(l2 - c), axis=0, keepdims=True)
    m2 = c + jnp.log2(s)
    vh = 2496
    l2a = jnp.dot(w_ref[:vh, :], xb, preferred_element_type=jnp.float32)
    l2b = jnp.dot(w_ref[vh:, :], xb, preferred_element_type=jnp.float32)
    o_ref[0, :vh, :] = (l2a - m2) * _LN2
    o_ref[0, vh:, :] = (l2b - m2) * _LN2


@jax.jit
def kernel(x, xl, W, b):
    B, T, D = x.shape
    V = W.shape[0]
    xb16 = x.astype(jnp.bfloat16).transpose(0, 2, 1)
    ones = jnp.ones((B, 1, T), dtype=jnp.bfloat16)
    zeros = jnp.zeros((B, _KA - D - 1, T), dtype=jnp.bfloat16)
    xt = jnp.concatenate([xb16, ones, zeros], axis=1)
    wb16 = (W * _LOG2E).astype(jnp.bfloat16)
    bcol = (b * _LOG2E).astype(jnp.bfloat16).reshape(V, 1)
    wzeros = jnp.zeros((V, _KA - D - 1), dtype=jnp.bfloat16)
    wa = jnp.concatenate([wb16, bcol, wzeros], axis=1)
    wf = wa.astype(jnp.float32)
    a_max = jnp.max(jnp.sqrt(jnp.sum(wf * wf, axis=1))).reshape(1, 1)

    out_t = pl.pallas_call(
        _logsoftmax_kernel,
        grid=(B, T // _R),
        in_specs=[
            pl.BlockSpec((1, _KA, _R), lambda bi, ti: (bi, 0, ti)),
            pl.BlockSpec((V, _KA), lambda bi, ti: (0, 0)),
            pl.BlockSpec((1, 1), lambda bi, ti: (0, 0)),
        ],
        out_specs=pl.BlockSpec((1, V, _R), lambda bi, ti: (bi, 0, ti)),
        out_shape=jax.ShapeDtypeStruct((B, V, T), jnp.float32),
    )(xt, wa, a_max)
    return out_t.transpose(0, 2, 1)


# speedup vs baseline: 1.0925x; 1.0237x over previous
import jax
import jax.numpy as jnp
from jax.experimental import pallas as pl

_R = 1024
_KA = 136
_LOG2E = 1.4426950408889634
_LN2 = 0.6931471805599453


def _logsoftmax_kernel(x_ref, w_ref, a_ref, o_ref):
    xb = x_ref[0]
    l2 = jnp.dot(w_ref[...], xb, preferred_element_type=jnp.float32)
    xf = xb.astype(jnp.float32)
    n2 = jnp.sum(xf * xf, axis=0, keepdims=True)
    c = a_ref[0, 0] * jnp.sqrt(n2) + 0.5
    s = jnp.sum(jnp.exp2(l2 - c), axis=0, keepdims=True)
    m2 = c + jnp.log2(s)
    vh = 2496
    l2a = jnp.dot(w_ref[:vh, :], xb, preferred_element_type=jnp.float32)
    l2b = jnp.dot(w_ref[vh:, :], xb, preferred_element_type=jnp.float32)
    o_ref[0, :vh, :] = (l2a - m2) * _LN2
    o_ref[0, vh:, :] = (l2b - m2) * _LN2


@jax.jit
def kernel(x, xl, W, b):
    B, T, D = x.shape
    V = W.shape[0]
    xb16 = x.astype(jnp.bfloat16).transpose(0, 2, 1)
    ones = jnp.ones((B, 1, T), dtype=jnp.bfloat16)
    zeros = jnp.zeros((B, _KA - D - 1, T), dtype=jnp.bfloat16)
    xt = jnp.concatenate([xb16, ones, zeros], axis=1)
    wb16 = (W * _LOG2E).astype(jnp.bfloat16)
    bcol = (b * _LOG2E).astype(jnp.bfloat16).reshape(V, 1)
    wzeros = jnp.zeros((V, _KA - D - 1), dtype=jnp.bfloat16)
    wa = jnp.concatenate([wb16, bcol, wzeros], axis=1)
    wf = wa.astype(jnp.float32)
    a_max = jnp.max(jnp.sqrt(jnp.sum(wf * wf, axis=1))).reshape(1, 1)

    out_t = pl.pallas_call(
        _logsoftmax_kernel,
        grid=(B, T // _R),
        in_specs=[
            pl.BlockSpec((1, _KA, _R), lambda bi, ti: (bi, 0, ti)),
            pl.BlockSpec((V, _KA), lambda bi, ti: (0, 0)),
            pl.BlockSpec((1, 1), lambda bi, ti: (0, 0)),
        ],
        out_specs=pl.BlockSpec((1, V, _R), lambda bi, ti: (bi, 0, ti)),
        out_shape=jax.ShapeDtypeStruct((B, V, T), jnp.float32),
    )(xt, wa, a_max)
    return out_t.transpose(0, 2, 1)


# speedup vs baseline: 1.1309x; 1.0352x over previous
import jax
import jax.numpy as jnp
from jax.experimental import pallas as pl

_R = 1024
_KA = 136
_LOG2E = 1.4426950408889634
_LN2 = 0.6931471805599453


def _dotT(w, xrows):
    return jax.lax.dot_general(
        w, xrows, (((1,), (1,)), ((), ())),
        preferred_element_type=jnp.float32,
    )


def _logsoftmax_kernel(x_ref, w_ref, a_ref, o_ref):
    xb = x_ref[0]
    l2 = _dotT(w_ref[...], xb)
    xf = xb.astype(jnp.float32)
    n2 = jnp.sum(xf * xf, axis=1, keepdims=True).T
    c = a_ref[0, 0] * jnp.sqrt(n2) + 0.5
    s = jnp.sum(jnp.exp2(l2 - c), axis=0, keepdims=True)
    m2 = c + jnp.log2(s)
    vh = 2496
    o_ref[0, :vh, :] = (_dotT(w_ref[:vh, :], xb) - m2) * _LN2
    o_ref[0, vh:, :] = (_dotT(w_ref[vh:, :], xb) - m2) * _LN2


@jax.jit
def kernel(x, xl, W, b):
    B, T, D = x.shape
    V = W.shape[0]
    xb16 = x.astype(jnp.bfloat16)
    ones = jnp.ones((B, T, 1), dtype=jnp.bfloat16)
    zeros = jnp.zeros((B, T, _KA - D - 1), dtype=jnp.bfloat16)
    xt = jnp.concatenate([xb16, ones, zeros], axis=2)
    wb16 = (W * _LOG2E).astype(jnp.bfloat16)
    bcol = (b * _LOG2E).astype(jnp.bfloat16).reshape(V, 1)
    wzeros = jnp.zeros((V, _KA - D - 1), dtype=jnp.bfloat16)
    wa = jnp.concatenate([wb16, bcol, wzeros], axis=1)
    wf = wa.astype(jnp.float32)
    a_max = jnp.max(jnp.sqrt(jnp.sum(wf * wf, axis=1))).reshape(1, 1)

    out_t = pl.pallas_call(
        _logsoftmax_kernel,
        grid=(B, T // _R),
        in_specs=[
            pl.BlockSpec((1, _R, _KA), lambda bi, ti: (bi, ti, 0)),
            pl.BlockSpec((V, _KA), lambda bi, ti: (0, 0)),
            pl.BlockSpec((1, 1), lambda bi, ti: (0, 0)),
        ],
        out_specs=pl.BlockSpec((1, V, _R), lambda bi, ti: (bi, 0, ti)),
        out_shape=jax.ShapeDtypeStruct((B, V, T), jnp.float32),
    )(xt, wa, a_max)
    return out_t.transpose(0, 2, 1)


# speedup vs baseline: 1.1933x; 1.0552x over previous
import jax
import jax.numpy as jnp
from jax.experimental import pallas as pl

_R = 1024
_KA = 136
_LOG2E = 1.4426950408889634
_LN2 = 0.6931471805599453


def _dotT(w, xrows):
    return jax.lax.dot_general(
        w, xrows, (((1,), (1,)), ((), ())),
        preferred_element_type=jnp.float32,
    )


def _logsoftmax_kernel(x_ref, w_ref, a_ref, o_ref):
    xf = x_ref[0]
    xb = jnp.concatenate(
        [xf.astype(jnp.bfloat16),
         jnp.ones((xf.shape[0], 1), dtype=jnp.bfloat16),
         jnp.zeros((xf.shape[0], _KA - 129), dtype=jnp.bfloat16)],
        axis=1,
    )
    l2 = _dotT(w_ref[...], xb)
    n2 = jnp.sum(xf * xf, axis=1, keepdims=True).T + 1.0
    c = a_ref[0, 0] * jnp.sqrt(n2) + 0.5
    s = jnp.sum(jnp.exp2(l2 - c), axis=0, keepdims=True)
    m2 = c + jnp.log2(s)
    vh = 2496
    o_ref[0, :vh, :] = (_dotT(w_ref[:vh, :], xb) - m2) * _LN2
    o_ref[0, vh:, :] = (_dotT(w_ref[vh:, :], xb) - m2) * _LN2


@jax.jit
def kernel(x, xl, W, b):
    B, T, D = x.shape
    V = W.shape[0]
    xt = x
    wb16 = (W * _LOG2E).astype(jnp.bfloat16)
    bcol = (b * _LOG2E).astype(jnp.bfloat16).reshape(V, 1)
    wzeros = jnp.zeros((V, _KA - D - 1), dtype=jnp.bfloat16)
    wa = jnp.concatenate([wb16, bcol, wzeros], axis=1)
    wf = wa.astype(jnp.float32)
    a_max = jnp.max(jnp.sqrt(jnp.sum(wf * wf, axis=1))).reshape(1, 1)

    out_t = pl.pallas_call(
        _logsoftmax_kernel,
        grid=(B, T // _R),
        in_specs=[
            pl.BlockSpec((1, _R, D), lambda bi, ti: (bi, ti, 0)),
            pl.BlockSpec((V, _KA), lambda bi, ti: (0, 0)),
            pl.BlockSpec((1, 1), lambda bi, ti: (0, 0)),
        ],
        out_specs=pl.BlockSpec((1, V, _R), lambda bi, ti: (bi, 0, ti)),
        out_shape=jax.ShapeDtypeStruct((B, V, T), jnp.float32),
    )(xt, wa, a_max)
    return out_t.transpose(0, 2, 1)
